# baseline (speedup 1.0000x reference)
import functools

import jax
import jax.numpy as jnp
from jax import lax
from jax.experimental import pallas as pl
from jax.experimental.pallas import tpu as pltpu
from jax.experimental.pallas import tpu_sc as plsc

N = 10000
E = 640000
ATOM = 101
BOND = 11
D = 128
L = 3
G = 256

NC = 2
NS = 16
NW = NC * NS
CH = 96
EPAD = 645120
TPT = EPAD // NW
NCHUNK = TPT // CH
NPAIR = NCHUNK // 2
NP = 10112
RPT = NP // NS
ZCHUNKS = [96] * 6 + [56]
F32 = jnp.float32


def _embed(x, w, b):
    def body(x_ref, w_ref, b_ref, o_ref):
        o_ref[:] = (
            jnp.dot(x_ref[:], w_ref[:], preferred_element_type=F32) + b_ref[:]
        )

    R = 1000
    return pl.pallas_call(
        body,
        grid=(N // R,),
        in_specs=[
            pl.BlockSpec((R, ATOM), lambda i: (i, 0)),
            pl.BlockSpec((ATOM, D), lambda i: (0, 0)),
            pl.BlockSpec((1, D), lambda i: (0, 0)),
        ],
        out_specs=pl.BlockSpec((R, D), lambda i: (i, 0)),
        out_shape=jax.ShapeDtypeStruct((N, D), F32),
    )(x, w, b)


def _bond(attr, w, b):
    B = 3840

    def body(a_ref, w_ref, b_ref, o_ref):
        o_ref[:] = (
            jnp.dot(a_ref[:], w_ref[:], preferred_element_type=F32) + b_ref[:]
        )

    return pl.pallas_call(
        body,
        grid=(EPAD // B,),
        in_specs=[
            pl.BlockSpec((B, BOND), lambda j: (j, 0)),
            pl.BlockSpec((BOND, D), lambda j: (0, 0)),
            pl.BlockSpec((1, D), lambda j: (0, 0)),
        ],
        out_specs=pl.BlockSpec((B, D), lambda j: (j, 0)),
        out_shape=jax.ShapeDtypeStruct((EPAD, D), F32),
    )(attr, w, b)


R = 1000
NB = N // R


def _mlp_fused(h, aggr, w1, b1, g1, be1, w2, b2, g2, be2, relu_last):
    def bn(v, g, be):
        m = jnp.mean(v, axis=0, keepdims=True)
        c = v - m
        var = jnp.mean(c * c, axis=0, keepdims=True)
        return c * lax.rsqrt(var + 1e-5) * g + be

    def body(h_ref, a_ref, w1_ref, b1_ref, g1_ref, be1_ref,
             w2_ref, b2_ref, g2_ref, be2_ref, o_ref):
        z = h_ref[:] + a_ref[0, :N] + a_ref[1, :N]
        z1 = jnp.dot(z, w1_ref[:], preferred_element_type=F32) + b1_ref[:]
        z1n = jnp.maximum(bn(z1, g1_ref[:], be1_ref[:]), 0.0)
        z2 = jnp.dot(z1n, w2_ref[:], preferred_element_type=F32) + b2_ref[:]
        hn = bn(z2, g2_ref[:], be2_ref[:])
        if relu_last:
            hn = jnp.maximum(hn, 0.0)
        o_ref[:] = hn

    return pl.pallas_call(
        body,
        out_shape=jax.ShapeDtypeStruct((N, D), F32),
    )(h, aggr, w1, b1, g1, be1, w2, b2, g2, be2)


def _bn_stats(vals, i, bm_ref, m2_ref):
    mb = jnp.mean(vals, axis=0, keepdims=True)
    c = vals - mb
    bm_ref[0] = mb
    m2 = jnp.sum(c * c, axis=0, keepdims=True)

    @pl.when(i == 0)
    def _():
        m2_ref[:] = jnp.zeros_like(m2_ref)

    m2_ref[:] += m2


def _bn_apply(vals, bm, m2, g, beta):
    mean = jnp.mean(bm, axis=0, keepdims=True)
    dm = bm - mean
    var = m2 * (1.0 / N) + jnp.mean(dm * dm, axis=0, keepdims=True)
    return (vals - mean) * lax.rsqrt(var + 1e-5) * g + beta


def _mlp1(h, aggr, w1, b1):
    def body(h_ref, a_ref, w_ref, b_ref, z_ref, bm_ref, m2_ref):
        z = h_ref[:] + a_ref[0] + a_ref[1]
        z1 = jnp.dot(z, w_ref[:], preferred_element_type=F32) + b_ref[:]
        z_ref[:] = z1
        _bn_stats(z1, pl.program_id(0), bm_ref, m2_ref)

    return pl.pallas_call(
        body,
        grid=(NB,),
        in_specs=[
            pl.BlockSpec((R, D), lambda i: (i, 0)),
            pl.BlockSpec((2, R, D), lambda i: (0, i, 0)),
            pl.BlockSpec((D, 2 * D), lambda i: (0, 0)),
            pl.BlockSpec((1, 2 * D), lambda i: (0, 0)),
        ],
        out_specs=[
            pl.BlockSpec((R, 2 * D), lambda i: (i, 0)),
            pl.BlockSpec((1, 1, 2 * D), lambda i: (i, 0, 0)),
            pl.BlockSpec((1, 2 * D), lambda i: (0, 0)),
        ],
        out_shape=[
            jax.ShapeDtypeStruct((N, 2 * D), F32),
            jax.ShapeDtypeStruct((NB, 1, 2 * D), F32),
            jax.ShapeDtypeStruct((1, 2 * D), F32),
        ],
    )(h, aggr, w1, b1)


def _mlp2(z1, bm1, m21, g, beta, w2, b2):
    def body(z_ref, bm1_ref, m21_ref, g_ref, be_ref, w_ref, b_ref,
             o_ref, bm_ref, m2_ref):
        zn = _bn_apply(z_ref[:], bm1_ref[:], m21_ref[:], g_ref[:], be_ref[:])
        r = jnp.maximum(zn, 0.0)
        z2 = jnp.dot(r, w_ref[:], preferred_element_type=F32) + b_ref[:]
        o_ref[:] = z2
        _bn_stats(z2, pl.program_id(0), bm_ref, m2_ref)

    return pl.pallas_call(
        body,
        grid=(NB,),
        in_specs=[
            pl.BlockSpec((R, 2 * D), lambda i: (i, 0)),
            pl.BlockSpec((NB, 2 * D), lambda i: (0, 0)),
            pl.BlockSpec((1, 2 * D), lambda i: (0, 0)),
            pl.BlockSpec((1, 2 * D), lambda i: (0, 0)),
            pl.BlockSpec((1, 2 * D), lambda i: (0, 0)),
            pl.BlockSpec((2 * D, D), lambda i: (0, 0)),
            pl.BlockSpec((1, D), lambda i: (0, 0)),
        ],
        out_specs=[
            pl.BlockSpec((R, D), lambda i: (i, 0)),
            pl.BlockSpec((1, 1, D), lambda i: (i, 0, 0)),
            pl.BlockSpec((1, D), lambda i: (0, 0)),
        ],
        out_shape=[
            jax.ShapeDtypeStruct((N, D), F32),
            jax.ShapeDtypeStruct((NB, 1, D), F32),
            jax.ShapeDtypeStruct((1, D), F32),
        ],
    )(z1, bm1, m21, g, beta, w2, b2)


def _mlp3(z2, bm2, m22, g, beta, relu):
    def body(z_ref, bm_ref, m2_ref, g_ref, be_ref, o_ref):
        hn = _bn_apply(z_ref[:], bm_ref[:], m2_ref[:], g_ref[:], be_ref[:])
        if relu:
            hn = jnp.maximum(hn, 0.0)
        o_ref[:] = hn

    return pl.pallas_call(
        body,
        grid=(NB,),
        in_specs=[
            pl.BlockSpec((R, D), lambda i: (i, 0)),
            pl.BlockSpec((NB, D), lambda i: (0, 0)),
            pl.BlockSpec((1, D), lambda i: (0, 0)),
            pl.BlockSpec((1, D), lambda i: (0, 0)),
            pl.BlockSpec((1, D), lambda i: (0, 0)),
        ],
        out_specs=pl.BlockSpec((R, D), lambda i: (i, 0)),
        out_shape=jax.ShapeDtypeStruct((N, D), F32),
    )(z2, bm2, m22, g, beta)


def _pool(h, batch2d):
    GB = 64

    def body(h_ref, b_ref, o_ref):
        gbase = pl.program_id(0) * GB
        rows = lax.broadcasted_iota(jnp.int32, (GB, N), 0) + gbase
        mask = (rows == b_ref[:]).astype(F32)
        sums = jnp.dot(mask, h_ref[:], preferred_element_type=F32)
        counts = jnp.sum(mask, axis=1, keepdims=True)
        o_ref[:] = sums / jnp.maximum(counts, 1.0)

    return pl.pallas_call(
        body,
        grid=(G // GB,),
        in_specs=[
            pl.BlockSpec((N, D), lambda i: (0, 0)),
            pl.BlockSpec((1, N), lambda i: (0, 0)),
        ],
        out_specs=pl.BlockSpec((GB, D), lambda i: (i, 0)),
        out_shape=jax.ShapeDtypeStruct((G, D), F32),
    )(h, batch2d)


def _messages(h, e, src2, dst2):
    mesh = plsc.VectorSubcoreMesh(core_axis_name="c", subcore_axis_name="s")

    @functools.partial(
        pl.kernel,
        mesh=mesh,
        out_type=jax.ShapeDtypeStruct((NC, NP, D), F32),
        scratch_types=[
            pltpu.VMEM((CH,), jnp.int32),
            pltpu.VMEM((CH,), jnp.int32),
            pltpu.VMEM((CH,), jnp.int32),
            pltpu.VMEM((CH,), jnp.int32),
            pltpu.VMEM((CH, D), F32),
            pltpu.VMEM((CH, D), F32),
            pltpu.VMEM((CH, D), F32),
            pltpu.VMEM((CH, D), F32),
            pltpu.VMEM_SHARED((NP, D), F32),
        ] + [pltpu.SemaphoreType.DMA] * 8,
    )
    def k(h_hbm, e_hbm, src_hbm, dst_hbm, out_hbm,
          sv0, dv0, sv1, dv1, h0, e0, h1, e1, aggr_sh,
          is0, id0, ig0, ie0, is1, id1, ig1, ie1):
        cid = lax.axis_index("c")
        sid = lax.axis_index("s")
        base = (cid * NS + sid) * TPT

        def zb(r, carry):
            for j in range(D // 16):
                e0[r, pl.ds(j * 16, 16)] = jnp.zeros((16,), F32)
            return carry

        lax.fori_loop(0, CH, zb, 0)
        zoff = 0
        for zc in ZCHUNKS:
            pltpu.sync_copy(e0.at[pl.ds(0, zc)],
                            aggr_sh.at[pl.ds(sid * RPT + zoff, zc)])
            zoff += zc
        plsc.subcore_barrier()

        def s_copy(c, sv, sem):
            off = base + c * CH
            return pltpu.make_async_copy(src_hbm.at[pl.ds(off, CH)], sv, sem)

        def d_copy(c, dv, sem):
            off = base + c * CH
            return pltpu.make_async_copy(dst_hbm.at[pl.ds(off, CH)], dv, sem)

        def e_copy(c, eb, sem):
            off = base + c * CH
            return pltpu.make_async_copy(e_hbm.at[pl.ds(off, CH)], eb, sem)

        def g_copy(sv, hb, sem):
            return pltpu.make_async_copy(h_hbm.at[sv], hb, sem)

        def process(p, c, sv, dv, hb, eb, s_is, s_id, s_ig, s_ie):
            g_copy(sv, hb, s_ig).wait()
            e_copy(c, eb, s_ie).wait()

            @pl.when(p < NPAIR - 1)
            def _():
                s_copy(c + 2, sv, s_is).start()

            def msg(r, c2):
                for j in range(D // 16):
                    s = pl.ds(j * 16, 16)
                    eb[r, s] = jnp.maximum(hb[r, s] + eb[r, s], 0.0)
                return c2

            lax.fori_loop(0, CH, msg, 0)

            @pl.when(p < NPAIR - 1)
            def _():
                s_copy(c + 2, sv, s_is).wait()
                g_copy(sv, hb, s_ig).start()

            d_copy(c, dv, s_id).wait()
            pltpu.sync_copy(eb, aggr_sh.at[dv], add=True)

            @pl.when(p < NPAIR - 1)
            def _():
                d_copy(c + 2, dv, s_id).start()
                e_copy(c + 2, eb, s_ie).start()

        s_copy(0, sv0, is0).start()
        d_copy(0, dv0, id0).start()
        e_copy(0, e0, ie0).start()
        s_copy(1, sv1, is1).start()
        d_copy(1, dv1, id1).start()
        e_copy(1, e1, ie1).start()
        s_copy(0, sv0, is0).wait()
        g_copy(sv0, h0, ig0).start()
        s_copy(1, sv1, is1).wait()
        g_copy(sv1, h1, ig1).start()

        def pair(p, carry):
            process(p, 2 * p, sv0, dv0, h0, e0, is0, id0, ig0, ie0)
            process(p, 2 * p + 1, sv1, dv1, h1, e1, is1, id1, ig1, ie1)
            return carry

        lax.fori_loop(0, NPAIR, pair, 0)
        plsc.subcore_barrier()
        zoff = 0
        for zc in ZCHUNKS:
            r0 = sid * RPT + zoff
            pltpu.sync_copy(aggr_sh.at[pl.ds(r0, zc)],
                            out_hbm.at[cid, pl.ds(r0, zc)])
            zoff += zc

    return k(h, e, src2, dst2)


def kernel(x, edge_index, edge_attr, batch, W_emb, b_emb, bond_W, bond_b,
           mlp_W1, mlp_b1, mlp_g, mlp_beta, mlp_W2, mlp_b2, bn_g, bn_beta):
    pad = EPAD - E
    src = jnp.concatenate([edge_index[0], jnp.zeros((pad,), jnp.int32)])
    dst = jnp.concatenate([edge_index[1], jnp.full((pad,), N, jnp.int32)])
    attr = jnp.concatenate(
        [edge_attr, jnp.zeros((pad, BOND), F32)], axis=0)

    h = _embed(x, W_emb, b_emb.reshape(1, D))
    e = _bond(attr, bond_W[0], bond_b[0].reshape(1, D))

    for i in range(L):
        aggr = _messages(h, e, src, dst)
        if i + 1 < L:
            e = _bond(attr, bond_W[i + 1], bond_b[i + 1].reshape(1, D))
        h = _mlp_fused(h, aggr, mlp_W1[i], mlp_b1[i].reshape(1, 2 * D),
                       mlp_g[i].reshape(1, 2 * D),
                       mlp_beta[i].reshape(1, 2 * D), mlp_W2[i],
                       mlp_b2[i].reshape(1, D), bn_g[i].reshape(1, D),
                       bn_beta[i].reshape(1, D), relu_last=(i < L - 1))

    return _pool(h, batch.reshape(1, N))


# speedup vs baseline: 1.1356x; 1.1356x over previous
import functools

import jax
import jax.numpy as jnp
from jax import lax
from jax.experimental import pallas as pl
from jax.experimental.pallas import tpu as pltpu
from jax.experimental.pallas import tpu_sc as plsc

N = 10000
E = 640000
ATOM = 101
BOND = 11
D = 128
L = 3
G = 256

NC = 2
NS = 16
NW = NC * NS
CH = 64
EPAD = 643072
TPT = EPAD // NW
NCHUNK = TPT // CH
NPAIR = NCHUNK // 2
NP = 10240
RPT = NP // NS
ZCHUNKS = [64] * 10
F32 = jnp.float32


def _embed(x, w, b):
    def body(x_ref, w_ref, b_ref, o_ref):
        o_ref[:] = (
            jnp.dot(x_ref[:], w_ref[:], preferred_element_type=F32) + b_ref[:]
        )

    R = 1000
    return pl.pallas_call(
        body,
        grid=(N // R,),
        in_specs=[
            pl.BlockSpec((R, ATOM), lambda i: (i, 0)),
            pl.BlockSpec((ATOM, D), lambda i: (0, 0)),
            pl.BlockSpec((1, D), lambda i: (0, 0)),
        ],
        out_specs=pl.BlockSpec((R, D), lambda i: (i, 0)),
        out_shape=jax.ShapeDtypeStruct((N, D), F32),
    )(x, w, b)


def _bond(attr, w, b):
    B = 4096

    def body(a_ref, w_ref, b_ref, o_ref):
        o_ref[:] = (
            jnp.dot(a_ref[:], w_ref[:], preferred_element_type=F32) + b_ref[:]
        )

    return pl.pallas_call(
        body,
        grid=(EPAD // B,),
        in_specs=[
            pl.BlockSpec((B, BOND), lambda j: (j, 0)),
            pl.BlockSpec((BOND, D), lambda j: (0, 0)),
            pl.BlockSpec((1, D), lambda j: (0, 0)),
        ],
        out_specs=pl.BlockSpec((B, D), lambda j: (j, 0)),
        out_shape=jax.ShapeDtypeStruct((EPAD, D), F32),
    )(attr, w, b)


R = 1000
NB = N // R


def _mlp_fused(h, aggr, w1, b1, g1, be1, w2, b2, g2, be2, relu_last):
    def bn(v, g, be):
        m = jnp.mean(v, axis=0, keepdims=True)
        c = v - m
        var = jnp.mean(c * c, axis=0, keepdims=True)
        return c * lax.rsqrt(var + 1e-5) * g + be

    def body(h_ref, a_ref, w1_ref, b1_ref, g1_ref, be1_ref,
             w2_ref, b2_ref, g2_ref, be2_ref, o_ref):
        z = h_ref[:] + a_ref[0, :N] + a_ref[1, :N]
        z1 = jnp.dot(z, w1_ref[:], preferred_element_type=F32) + b1_ref[:]
        z1n = jnp.maximum(bn(z1, g1_ref[:], be1_ref[:]), 0.0)
        z2 = jnp.dot(z1n, w2_ref[:], preferred_element_type=F32) + b2_ref[:]
        hn = bn(z2, g2_ref[:], be2_ref[:])
        if relu_last:
            hn = jnp.maximum(hn, 0.0)
        o_ref[:] = hn

    return pl.pallas_call(
        body,
        out_shape=jax.ShapeDtypeStruct((N, D), F32),
    )(h, aggr, w1, b1, g1, be1, w2, b2, g2, be2)


def _bn_stats(vals, i, bm_ref, m2_ref):
    mb = jnp.mean(vals, axis=0, keepdims=True)
    c = vals - mb
    bm_ref[0] = mb
    m2 = jnp.sum(c * c, axis=0, keepdims=True)

    @pl.when(i == 0)
    def _():
        m2_ref[:] = jnp.zeros_like(m2_ref)

    m2_ref[:] += m2


def _bn_apply(vals, bm, m2, g, beta):
    mean = jnp.mean(bm, axis=0, keepdims=True)
    dm = bm - mean
    var = m2 * (1.0 / N) + jnp.mean(dm * dm, axis=0, keepdims=True)
    return (vals - mean) * lax.rsqrt(var + 1e-5) * g + beta


def _mlp1(h, aggr, w1, b1):
    def body(h_ref, a_ref, w_ref, b_ref, z_ref, bm_ref, m2_ref):
        z = h_ref[:] + a_ref[0] + a_ref[1]
        z1 = jnp.dot(z, w_ref[:], preferred_element_type=F32) + b_ref[:]
        z_ref[:] = z1
        _bn_stats(z1, pl.program_id(0), bm_ref, m2_ref)

    return pl.pallas_call(
        body,
        grid=(NB,),
        in_specs=[
            pl.BlockSpec((R, D), lambda i: (i, 0)),
            pl.BlockSpec((2, R, D), lambda i: (0, i, 0)),
            pl.BlockSpec((D, 2 * D), lambda i: (0, 0)),
            pl.BlockSpec((1, 2 * D), lambda i: (0, 0)),
        ],
        out_specs=[
            pl.BlockSpec((R, 2 * D), lambda i: (i, 0)),
            pl.BlockSpec((1, 1, 2 * D), lambda i: (i, 0, 0)),
            pl.BlockSpec((1, 2 * D), lambda i: (0, 0)),
        ],
        out_shape=[
            jax.ShapeDtypeStruct((N, 2 * D), F32),
            jax.ShapeDtypeStruct((NB, 1, 2 * D), F32),
            jax.ShapeDtypeStruct((1, 2 * D), F32),
        ],
    )(h, aggr, w1, b1)


def _mlp2(z1, bm1, m21, g, beta, w2, b2):
    def body(z_ref, bm1_ref, m21_ref, g_ref, be_ref, w_ref, b_ref,
             o_ref, bm_ref, m2_ref):
        zn = _bn_apply(z_ref[:], bm1_ref[:], m21_ref[:], g_ref[:], be_ref[:])
        r = jnp.maximum(zn, 0.0)
        z2 = jnp.dot(r, w_ref[:], preferred_element_type=F32) + b_ref[:]
        o_ref[:] = z2
        _bn_stats(z2, pl.program_id(0), bm_ref, m2_ref)

    return pl.pallas_call(
        body,
        grid=(NB,),
        in_specs=[
            pl.BlockSpec((R, 2 * D), lambda i: (i, 0)),
            pl.BlockSpec((NB, 2 * D), lambda i: (0, 0)),
            pl.BlockSpec((1, 2 * D), lambda i: (0, 0)),
            pl.BlockSpec((1, 2 * D), lambda i: (0, 0)),
            pl.BlockSpec((1, 2 * D), lambda i: (0, 0)),
            pl.BlockSpec((2 * D, D), lambda i: (0, 0)),
            pl.BlockSpec((1, D), lambda i: (0, 0)),
        ],
        out_specs=[
            pl.BlockSpec((R, D), lambda i: (i, 0)),
            pl.BlockSpec((1, 1, D), lambda i: (i, 0, 0)),
            pl.BlockSpec((1, D), lambda i: (0, 0)),
        ],
        out_shape=[
            jax.ShapeDtypeStruct((N, D), F32),
            jax.ShapeDtypeStruct((NB, 1, D), F32),
            jax.ShapeDtypeStruct((1, D), F32),
        ],
    )(z1, bm1, m21, g, beta, w2, b2)


def _mlp3(z2, bm2, m22, g, beta, relu):
    def body(z_ref, bm_ref, m2_ref, g_ref, be_ref, o_ref):
        hn = _bn_apply(z_ref[:], bm_ref[:], m2_ref[:], g_ref[:], be_ref[:])
        if relu:
            hn = jnp.maximum(hn, 0.0)
        o_ref[:] = hn

    return pl.pallas_call(
        body,
        grid=(NB,),
        in_specs=[
            pl.BlockSpec((R, D), lambda i: (i, 0)),
            pl.BlockSpec((NB, D), lambda i: (0, 0)),
            pl.BlockSpec((1, D), lambda i: (0, 0)),
            pl.BlockSpec((1, D), lambda i: (0, 0)),
            pl.BlockSpec((1, D), lambda i: (0, 0)),
        ],
        out_specs=pl.BlockSpec((R, D), lambda i: (i, 0)),
        out_shape=jax.ShapeDtypeStruct((N, D), F32),
    )(z2, bm2, m22, g, beta)


def _pool(h, batch2d):
    GB = 64

    def body(h_ref, b_ref, o_ref):
        gbase = pl.program_id(0) * GB
        rows = lax.broadcasted_iota(jnp.int32, (GB, N), 0) + gbase
        mask = (rows == b_ref[:]).astype(F32)
        sums = jnp.dot(mask, h_ref[:], preferred_element_type=F32)
        counts = jnp.sum(mask, axis=1, keepdims=True)
        o_ref[:] = sums / jnp.maximum(counts, 1.0)

    return pl.pallas_call(
        body,
        grid=(G // GB,),
        in_specs=[
            pl.BlockSpec((N, D), lambda i: (0, 0)),
            pl.BlockSpec((1, N), lambda i: (0, 0)),
        ],
        out_specs=pl.BlockSpec((GB, D), lambda i: (i, 0)),
        out_shape=jax.ShapeDtypeStruct((G, D), F32),
    )(h, batch2d)


def _messages(h, e, src2, dst2):
    mesh = plsc.VectorSubcoreMesh(core_axis_name="c", subcore_axis_name="s")

    @functools.partial(
        pl.kernel,
        mesh=mesh,
        out_type=jax.ShapeDtypeStruct((NC, NP, D), F32),
        scratch_types=[
            pltpu.VMEM((CH,), jnp.int32),
            pltpu.VMEM((CH,), jnp.int32),
            pltpu.VMEM((CH,), jnp.int32),
            pltpu.VMEM((CH,), jnp.int32),
            pltpu.VMEM((CH, D), F32),
            pltpu.VMEM((CH, D), F32),
            pltpu.VMEM((CH, D), F32),
            pltpu.VMEM((CH, D), F32),
            pltpu.VMEM_SHARED((NP, D), F32),
        ] + [pltpu.SemaphoreType.DMA] * 8,
    )
    def k(h_hbm, e_hbm, src_hbm, dst_hbm, out_hbm,
          sv0, dv0, sv1, dv1, h0, e0, h1, e1, aggr_sh,
          is0, id0, ig0, ie0, is1, id1, ig1, ie1):
        cid = lax.axis_index("c")
        sid = lax.axis_index("s")
        base = (cid * NS + sid) * TPT

        def zb(r, carry):
            for j in range(D // 16):
                e0[r, pl.ds(j * 16, 16)] = jnp.zeros((16,), F32)
            return carry

        lax.fori_loop(0, CH, zb, 0)
        zoff = 0
        for zc in ZCHUNKS:
            pltpu.sync_copy(e0.at[pl.ds(0, zc)],
                            aggr_sh.at[pl.ds(sid * RPT + zoff, zc)])
            zoff += zc
        plsc.subcore_barrier()

        def s_copy(c, sv, sem):
            off = base + c * CH
            return pltpu.make_async_copy(src_hbm.at[pl.ds(off, CH)], sv, sem)

        def d_copy(c, dv, sem):
            off = base + c * CH
            return pltpu.make_async_copy(dst_hbm.at[pl.ds(off, CH)], dv, sem)

        def e_copy(c, eb, sem):
            off = base + c * CH
            return pltpu.make_async_copy(e_hbm.at[pl.ds(off, CH)], eb, sem)

        def g_copy(sv, hb, sem):
            return pltpu.make_async_copy(h_hbm.at[sv], hb, sem)

        def process(p, c, sv, dv, hb, eb, s_is, s_id, s_ig, s_ie):
            g_copy(sv, hb, s_ig).wait()
            e_copy(c, eb, s_ie).wait()

            @pl.when(p < NPAIR - 1)
            def _():
                s_copy(c + 2, sv, s_is).start()

            def msg(r, c2):
                for j in range(D // 16):
                    s = pl.ds(j * 16, 16)
                    eb[r, s] = jnp.maximum(hb[r, s] + eb[r, s], 0.0)
                return c2

            lax.fori_loop(0, CH, msg, 0)

            @pl.when(p < NPAIR - 1)
            def _():
                s_copy(c + 2, sv, s_is).wait()
                g_copy(sv, hb, s_ig).start()

            d_copy(c, dv, s_id).wait()
            pltpu.sync_copy(eb, aggr_sh.at[dv], add=True)

            @pl.when(p < NPAIR - 1)
            def _():
                d_copy(c + 2, dv, s_id).start()
                e_copy(c + 2, eb, s_ie).start()

        s_copy(0, sv0, is0).start()
        d_copy(0, dv0, id0).start()
        e_copy(0, e0, ie0).start()
        s_copy(1, sv1, is1).start()
        d_copy(1, dv1, id1).start()
        e_copy(1, e1, ie1).start()
        s_copy(0, sv0, is0).wait()
        g_copy(sv0, h0, ig0).start()
        s_copy(1, sv1, is1).wait()
        g_copy(sv1, h1, ig1).start()

        def pair(p, carry):
            process(p, 2 * p, sv0, dv0, h0, e0, is0, id0, ig0, ie0)
            process(p, 2 * p + 1, sv1, dv1, h1, e1, is1, id1, ig1, ie1)
            return carry

        lax.fori_loop(0, NPAIR, pair, 0)
        plsc.subcore_barrier()
        zoff = 0
        for zc in ZCHUNKS:
            r0 = sid * RPT + zoff
            pltpu.sync_copy(aggr_sh.at[pl.ds(r0, zc)],
                            out_hbm.at[cid, pl.ds(r0, zc)])
            zoff += zc

    return k(h, e, src2, dst2)


def kernel(x, edge_index, edge_attr, batch, W_emb, b_emb, bond_W, bond_b,
           mlp_W1, mlp_b1, mlp_g, mlp_beta, mlp_W2, mlp_b2, bn_g, bn_beta):
    pad = EPAD - E
    src = jnp.concatenate([edge_index[0], jnp.zeros((pad,), jnp.int32)])
    dst = jnp.concatenate([edge_index[1], jnp.full((pad,), N, jnp.int32)])
    attr = jnp.concatenate(
        [edge_attr, jnp.zeros((pad, BOND), F32)], axis=0)

    h = _embed(x, W_emb, b_emb.reshape(1, D))
    e = _bond(attr, bond_W[0], bond_b[0].reshape(1, D))

    for i in range(L):
        aggr = _messages(h, e, src, dst)
        if i + 1 < L:
            e = _bond(attr, bond_W[i + 1], bond_b[i + 1].reshape(1, D))
        h = _mlp_fused(h, aggr, mlp_W1[i], mlp_b1[i].reshape(1, 2 * D),
                       mlp_g[i].reshape(1, 2 * D),
                       mlp_beta[i].reshape(1, 2 * D), mlp_W2[i],
                       mlp_b2[i].reshape(1, D), bn_g[i].reshape(1, D),
                       bn_beta[i].reshape(1, D), relu_last=(i < L - 1))

    return _pool(h, batch.reshape(1, N))
